# baseline (speedup 1.0000x reference)
import functools

import jax
import jax.numpy as jnp
from jax import lax
from jax.experimental import pallas as pl
from jax.experimental.pallas import tpu as pltpu
from jax.experimental.pallas import tpu_sc as plsc

VOCAB_SIZES = (6, 7, 12, 7, 96)
EMB_DIMS = (3, 4, 6, 4, 50)
N_ROWS = 16384
X_COLS = 64
OUT_COLS = X_COLS + sum(EMB_DIMS)

NUM_CORES = 2
NUM_SUBCORES = 16
NUM_WORKERS = NUM_CORES * NUM_SUBCORES
COLS_PER_W = N_ROWS // NUM_WORKERS
NSB = 1
SB_COLS = COLS_PER_W // NSB
LANES = 16
SB_CHUNKS = SB_COLS // LANES

_COL_OFF = []
_acc = X_COLS
for _d in EMB_DIMS:
    _COL_OFF.append(_acc)
    _acc += _d


def _body(xt_hbm, yt_hbm, e0, e1, e2, e3, e4, out_hbm, y_v, t0, t1, t2, t3, t4,
          o_v0, sem, xsem0, osem0):
    tabs = (t0, t1, t2, t3, t4)
    o_bufs = (o_v0,)
    xsems = (xsem0,)
    osems = (osem0,)
    wid = lax.axis_index("s") * NUM_CORES + lax.axis_index("c")
    base = wid * COLS_PER_W

    xcps = [
        pltpu.make_async_copy(
            xt_hbm.at[:, pl.ds(base + sb * SB_COLS, SB_COLS)],
            o_bufs[sb].at[pl.ds(0, X_COLS), :], xsems[sb])
        for sb in range(NSB)
    ]
    for cp in xcps:
        cp.start()
    cps = [
        pltpu.make_async_copy(yt_hbm.at[:, pl.ds(base, COLS_PER_W)], y_v, sem),
    ] + [
        pltpu.make_async_copy(e, t, sem)
        for e, t in zip((e0, e1, e2, e3, e4), tabs)
    ]
    for cp in cps:
        cp.start()
    o_v = o_bufs[0]
    xcps[0].wait()
    ocp_x = pltpu.make_async_copy(
        o_v.at[pl.ds(0, X_COLS), :],
        out_hbm.at[pl.ds(0, X_COLS), pl.ds(base, COLS_PER_W)], xsems[0])
    ocp_x.start()
    for cp in cps:
        cp.wait()

    @plsc.parallel_loop(0, SB_COLS, step=LANES, unroll=6)
    def _chunk(s):
        for i in range(5):
            yi = y_v[i, pl.ds(s, LANES)]
            for cc in range(EMB_DIMS[i]):
                val = plsc.load_gather(
                    tabs[i], [jnp.full((LANES,), cc, jnp.int32), yi])
                o_v[_COL_OFF[i] + cc, pl.ds(s, LANES)] = val

    ocp_e = pltpu.make_async_copy(
        o_v.at[pl.ds(X_COLS, OUT_COLS - X_COLS), :],
        out_hbm.at[pl.ds(X_COLS, OUT_COLS - X_COLS), pl.ds(base, COLS_PER_W)],
        osems[0])
    ocp_e.start()
    ocp_x.wait()
    ocp_e.wait()


_feature_call = functools.partial(
    pl.kernel,
    out_type=jax.ShapeDtypeStruct((OUT_COLS, N_ROWS), jnp.float32),
    mesh=plsc.VectorSubcoreMesh(core_axis_name="c", subcore_axis_name="s"),
    compiler_params=pltpu.CompilerParams(
        needs_layout_passes=False, use_tc_tiling_on_sc=True),
    scratch_types=[
        pltpu.VMEM((5, COLS_PER_W), jnp.int32),
        pltpu.VMEM((EMB_DIMS[0], VOCAB_SIZES[0]), jnp.float32),
        pltpu.VMEM((EMB_DIMS[1], VOCAB_SIZES[1]), jnp.float32),
        pltpu.VMEM((EMB_DIMS[2], VOCAB_SIZES[2]), jnp.float32),
        pltpu.VMEM((EMB_DIMS[3], VOCAB_SIZES[3]), jnp.float32),
        pltpu.VMEM((EMB_DIMS[4], VOCAB_SIZES[4]), jnp.float32),
        pltpu.VMEM((OUT_COLS, SB_COLS), jnp.float32),
        pltpu.SemaphoreType.DMA,
        pltpu.SemaphoreType.DMA,
        pltpu.SemaphoreType.DMA,
    ],
)(_body)


def kernel(x, y, E0, E1, E2, E3, E4):
    out_t = _feature_call(x.T, y.T, E0.T, E1.T, E2.T, E3.T, E4.T)
    return out_t.T


# speedup vs baseline: 1.0274x; 1.0274x over previous
import functools

import jax
import jax.numpy as jnp
from jax import lax
from jax.experimental import pallas as pl
from jax.experimental.pallas import tpu as pltpu
from jax.experimental.pallas import tpu_sc as plsc

VOCAB_SIZES = (6, 7, 12, 7, 96)
EMB_DIMS = (3, 4, 6, 4, 50)
N_ROWS = 16384
X_COLS = 64
OUT_COLS = X_COLS + sum(EMB_DIMS)

NUM_CORES = 2
NUM_SUBCORES = 16
NUM_WORKERS = NUM_CORES * NUM_SUBCORES
COLS_PER_W = N_ROWS // NUM_WORKERS
NSB = 1
SB_COLS = COLS_PER_W // NSB
LANES = 16
SB_CHUNKS = SB_COLS // LANES

_COL_OFF = []
_acc = X_COLS
for _d in EMB_DIMS:
    _COL_OFF.append(_acc)
    _acc += _d


def _body(xt_hbm, yt_hbm, e0, e1, e2, e3, e4, out_hbm, y_v, t0, t1, t2, t3, t4,
          o_v0, sem, xsem0, osem0):
    tabs = (t0, t1, t2, t3, t4)
    o_bufs = (o_v0,)
    xsems = (xsem0,)
    osems = (osem0,)
    wid = lax.axis_index("s") * NUM_CORES + lax.axis_index("c")
    base = wid * COLS_PER_W

    xcps = [
        pltpu.make_async_copy(
            xt_hbm.at[:, pl.ds(base + sb * SB_COLS, SB_COLS)],
            o_bufs[sb].at[pl.ds(0, X_COLS), :], xsems[sb])
        for sb in range(NSB)
    ]
    for cp in xcps:
        cp.start()
    cps = [
        pltpu.make_async_copy(yt_hbm.at[:, pl.ds(base, COLS_PER_W)], y_v, sem),
    ] + [
        pltpu.make_async_copy(e, t, sem)
        for e, t in zip((e0, e1, e2, e3, e4), tabs)
    ]
    for cp in cps:
        cp.start()
    o_v = o_bufs[0]
    xcps[0].wait()
    ocp_x = pltpu.make_async_copy(
        o_v.at[pl.ds(0, X_COLS), :],
        out_hbm.at[pl.ds(0, X_COLS), pl.ds(base, COLS_PER_W)], xsems[0])
    ocp_x.start()
    for cp in cps:
        cp.wait()

    @plsc.parallel_loop(0, SB_COLS, step=LANES, unroll=4)
    def _chunk(s):
        for i in range(5):
            yi = y_v[i, pl.ds(s, LANES)]
            for cc in range(EMB_DIMS[i]):
                val = plsc.load_gather(
                    tabs[i], [jnp.full((LANES,), cc, jnp.int32), yi])
                o_v[_COL_OFF[i] + cc, pl.ds(s, LANES)] = val

    ocp_e = pltpu.make_async_copy(
        o_v.at[pl.ds(X_COLS, OUT_COLS - X_COLS), :],
        out_hbm.at[pl.ds(X_COLS, OUT_COLS - X_COLS), pl.ds(base, COLS_PER_W)],
        osems[0])
    ocp_e.start()
    ocp_x.wait()
    ocp_e.wait()


_feature_call = functools.partial(
    pl.kernel,
    out_type=jax.ShapeDtypeStruct((OUT_COLS, N_ROWS), jnp.float32),
    mesh=plsc.VectorSubcoreMesh(core_axis_name="c", subcore_axis_name="s"),
    compiler_params=pltpu.CompilerParams(
        needs_layout_passes=False, use_tc_tiling_on_sc=True,
        skip_device_barrier=True, disable_bounds_checks=True,
        disable_semaphore_checks=True),
    scratch_types=[
        pltpu.VMEM((5, COLS_PER_W), jnp.int32),
        pltpu.VMEM((EMB_DIMS[0], VOCAB_SIZES[0]), jnp.float32),
        pltpu.VMEM((EMB_DIMS[1], VOCAB_SIZES[1]), jnp.float32),
        pltpu.VMEM((EMB_DIMS[2], VOCAB_SIZES[2]), jnp.float32),
        pltpu.VMEM((EMB_DIMS[3], VOCAB_SIZES[3]), jnp.float32),
        pltpu.VMEM((EMB_DIMS[4], VOCAB_SIZES[4]), jnp.float32),
        pltpu.VMEM((OUT_COLS, SB_COLS), jnp.float32),
        pltpu.SemaphoreType.DMA,
        pltpu.SemaphoreType.DMA,
        pltpu.SemaphoreType.DMA,
    ],
)(_body)


def kernel(x, y, E0, E1, E2, E3, E4):
    out_t = _feature_call(x.T, y.T, E0.T, E1.T, E2.T, E3.T, E4.T)
    return out_t.T


# speedup vs baseline: 1.0766x; 1.0479x over previous
import functools

import jax
import jax.numpy as jnp
from jax import lax
from jax.experimental import pallas as pl
from jax.experimental.pallas import tpu as pltpu
from jax.experimental.pallas import tpu_sc as plsc

VOCAB_SIZES = (6, 7, 12, 7, 96)
EMB_DIMS = (3, 4, 6, 4, 50)
N_ROWS = 16384
X_COLS = 64
OUT_COLS = X_COLS + sum(EMB_DIMS)

NUM_CORES = 2
NUM_SUBCORES = 16
NUM_WORKERS = NUM_CORES * NUM_SUBCORES
COLS_PER_W = N_ROWS // NUM_WORKERS
NSB = 1
SB_COLS = COLS_PER_W // NSB
LANES = 16
SB_CHUNKS = SB_COLS // LANES

_COL_OFF = []
_acc = X_COLS
for _d in EMB_DIMS:
    _COL_OFF.append(_acc)
    _acc += _d


def _body(xt_hbm, yt_hbm, e0, e1, e2, e3, e4, out_hbm, y_v, t0, t1, t2, t3, t4,
          o_v0, sem, xsem0, osem0):
    tabs = (t0, t1, t2, t3, t4)
    o_bufs = (o_v0,)
    xsems = (xsem0,)
    osems = (osem0,)
    wid = lax.axis_index("s") * NUM_CORES + lax.axis_index("c")
    base = wid * COLS_PER_W

    xcps = [
        pltpu.make_async_copy(
            xt_hbm.at[:, pl.ds(base + sb * SB_COLS, SB_COLS)],
            o_bufs[sb].at[pl.ds(0, X_COLS), :], xsems[sb])
        for sb in range(NSB)
    ]
    for cp in xcps:
        cp.start()
    cps = [
        pltpu.make_async_copy(yt_hbm.at[:, pl.ds(base, COLS_PER_W)], y_v, sem),
    ] + [
        pltpu.make_async_copy(e, t, sem)
        for e, t in zip((e0, e1, e2, e3, e4), tabs)
    ]
    for cp in cps:
        cp.start()
    o_v = o_bufs[0]
    xcps[0].wait()
    ocp_x = pltpu.make_async_copy(
        o_v.at[pl.ds(0, X_COLS), :],
        out_hbm.at[pl.ds(0, X_COLS), pl.ds(base, COLS_PER_W)], xsems[0])
    ocp_x.start()
    for cp in cps:
        cp.wait()

    @plsc.parallel_loop(0, SB_COLS, step=LANES, unroll=1)
    def _chunk(s):
        for i in range(5):
            yi = y_v[i, pl.ds(s, LANES)]
            for cc in range(EMB_DIMS[i]):
                val = plsc.load_gather(
                    tabs[i], [jnp.full((LANES,), cc, jnp.int32), yi])
                o_v[_COL_OFF[i] + cc, pl.ds(s, LANES)] = val

    ocp_e = pltpu.make_async_copy(
        o_v.at[pl.ds(X_COLS, OUT_COLS - X_COLS), :],
        out_hbm.at[pl.ds(X_COLS, OUT_COLS - X_COLS), pl.ds(base, COLS_PER_W)],
        osems[0])
    ocp_e.start()
    ocp_x.wait()
    ocp_e.wait()


_feature_call = functools.partial(
    pl.kernel,
    out_type=jax.ShapeDtypeStruct((OUT_COLS, N_ROWS), jnp.float32),
    mesh=plsc.VectorSubcoreMesh(core_axis_name="c", subcore_axis_name="s"),
    compiler_params=pltpu.CompilerParams(
        needs_layout_passes=False, use_tc_tiling_on_sc=True),
    scratch_types=[
        pltpu.VMEM((5, COLS_PER_W), jnp.int32),
        pltpu.VMEM((EMB_DIMS[0], VOCAB_SIZES[0]), jnp.float32),
        pltpu.VMEM((EMB_DIMS[1], VOCAB_SIZES[1]), jnp.float32),
        pltpu.VMEM((EMB_DIMS[2], VOCAB_SIZES[2]), jnp.float32),
        pltpu.VMEM((EMB_DIMS[3], VOCAB_SIZES[3]), jnp.float32),
        pltpu.VMEM((EMB_DIMS[4], VOCAB_SIZES[4]), jnp.float32),
        pltpu.VMEM((OUT_COLS, SB_COLS), jnp.float32),
        pltpu.SemaphoreType.DMA,
        pltpu.SemaphoreType.DMA,
        pltpu.SemaphoreType.DMA,
    ],
)(_body)


def kernel(x, y, E0, E1, E2, E3, E4):
    out_t = _feature_call(x.T, y.T, E0.T, E1.T, E2.T, E3.T, E4.T)
    return out_t.T
